# baseline (speedup 1.0000x reference)
import functools

import jax
import jax.numpy as jnp
from jax import lax
from jax.experimental import pallas as pl
from jax.experimental.pallas import tpu as pltpu
from jax.experimental.pallas import tpu_sc as plsc

VOCAB = 1000000
DIM = 64
BATCH = 16384
FIELDS = 26

_INFO = plsc.get_sparse_core_info()
_NC = _INFO.num_cores
_NS = _INFO.num_subcores
_NW = _NC * _NS

_B = BATCH * FIELDS
_BPW = _B // _NW
_CHUNK = 512
_SUB = 128
_NSUB = _CHUNK // _SUB
_NITER = _BPW // _CHUNK


def _make_kernel():
    mesh = plsc.VectorSubcoreMesh(core_axis_name="c", subcore_axis_name="s")

    @functools.partial(
        pl.kernel,
        mesh=mesh,
        out_type=jax.ShapeDtypeStruct((_B, DIM), jnp.float32),
        scratch_types=[
            pltpu.VMEM((_BPW,), jnp.int32),
            pltpu.VMEM((_CHUNK, DIM), jnp.float32),
            pltpu.SemaphoreType.DMA,
        ],
        compiler_params=pltpu.CompilerParams(use_tc_tiling_on_sc=False),
    )
    def emb(x_hbm, table_hbm, out_hbm, idx_v, rows_v, sem):
        wid = lax.axis_index("s") * _NC + lax.axis_index("c")
        base = wid * _BPW
        pltpu.sync_copy(x_hbm.at[pl.ds(base, _BPW)], idx_v)

        def body(i, carry):
            cps = []
            for g in range(_NSUB):
                cps.append(pltpu.async_copy(
                    table_hbm.at[idx_v.at[pl.ds(i * _CHUNK + g * _SUB, _SUB)]],
                    rows_v.at[pl.ds(g * _SUB, _SUB)],
                    sem,
                ))
            for cp in cps:
                cp.wait()
            pltpu.sync_copy(rows_v, out_hbm.at[pl.ds(base + i * _CHUNK, _CHUNK)])
            return carry

        lax.fori_loop(0, _NITER, body, 0)

    return emb


_EMB = _make_kernel()


def kernel(x, weight):
    out = _EMB(x.reshape(-1).astype(jnp.int32), weight)
    return out.reshape(BATCH, FIELDS, DIM)


# speedup vs baseline: 1.0165x; 1.0165x over previous
import functools

import jax
import jax.numpy as jnp
from jax import lax
from jax.experimental import pallas as pl
from jax.experimental.pallas import tpu as pltpu
from jax.experimental.pallas import tpu_sc as plsc

VOCAB = 1000000
DIM = 64
BATCH = 16384
FIELDS = 26

_INFO = plsc.get_sparse_core_info()
_NC = _INFO.num_cores
_NS = _INFO.num_subcores
_NW = _NC * _NS

_B = BATCH * FIELDS
_BPW = _B // _NW
_CHUNK = 256
_SUB = 128
_NSUB = _CHUNK // _SUB
_NITER = _BPW // _CHUNK
_NB = 4
_NGROUP = _NITER // _NB


def _make_kernel():
    mesh = plsc.VectorSubcoreMesh(core_axis_name="c", subcore_axis_name="s")

    @functools.partial(
        pl.kernel,
        mesh=mesh,
        out_type=jax.ShapeDtypeStruct((_B, DIM), jnp.float32),
        scratch_types=[
            pltpu.VMEM((_BPW,), jnp.int32),
            pltpu.VMEM((_NB, _CHUNK, DIM), jnp.float32),
        ] + [pltpu.SemaphoreType.DMA] * (2 * _NB),
        compiler_params=pltpu.CompilerParams(use_tc_tiling_on_sc=False),
    )
    def emb(x_hbm, table_hbm, out_hbm, idx_v, rows_v, *sems):
        gsems, osems = sems[:_NB], sems[_NB:]
        wid = lax.axis_index("s") * _NC + lax.axis_index("c")
        base = wid * _BPW
        pltpu.sync_copy(x_hbm.at[pl.ds(base, _BPW)], idx_v)

        def fire_gather(c, b):
            for g in range(_NSUB):
                pltpu.async_copy(
                    table_hbm.at[idx_v.at[pl.ds(c * _CHUNK + g * _SUB, _SUB)]],
                    rows_v.at[b].at[pl.ds(g * _SUB, _SUB)],
                    gsems[b],
                )

        def wait_gather(b):
            for g in range(_NSUB):
                pltpu.make_async_copy(
                    table_hbm.at[idx_v.at[pl.ds(g * _SUB, _SUB)]],
                    rows_v.at[b].at[pl.ds(g * _SUB, _SUB)],
                    gsems[b],
                ).wait()

        def wait_out(b):
            pltpu.make_async_copy(
                rows_v.at[b],
                out_hbm.at[pl.ds(base, _CHUNK)],
                osems[b],
            ).wait()

        for b in range(_NB):
            fire_gather(b, b)

        def body(j, carry):
            for b in range(_NB):
                c = j * _NB + b
                wait_gather(b)
                pltpu.async_copy(
                    rows_v.at[b],
                    out_hbm.at[pl.ds(base + c * _CHUNK, _CHUNK)],
                    osems[b],
                )
                bp = (b - 1) % _NB
                cc = c - 1 + _NB
                if b == 0:
                    @pl.when(j >= 1)
                    def _():
                        wait_out(bp)
                        fire_gather(cc, bp)
                else:
                    wait_out(bp)
                    @pl.when(j < _NGROUP - 1)
                    def _():
                        fire_gather(cc, bp)
            return carry

        lax.fori_loop(0, _NGROUP, body, 0)
        wait_out(_NB - 1)

    return emb


_EMB = _make_kernel()


def kernel(x, weight):
    out = _EMB(x.reshape(-1).astype(jnp.int32), weight)
    return out.reshape(BATCH, FIELDS, DIM)
